# Initial kernel scaffold; baseline (speedup 1.0000x reference)
import jax
import jax.numpy as jnp
from jax.experimental import pallas as pl


def kernel(inputs, word_table, pos_table):
    raise NotImplementedError("write your pallas kernel here")


# speedup vs baseline: 3.0651x; 3.0651x over previous
import functools

import jax
import jax.numpy as jnp
from jax import lax
from jax.experimental import pallas as pl
from jax.experimental.pallas import tpu as pltpu
from jax.experimental.pallas import tpu_sc as plsc

MAX_WORD = 100000
EMBED_DIM = 64
SEQ_LENGTH = 200
BATCH = 4096

NUM_CORES = 2
NUM_SUBCORES = 16
NUM_WORKERS = NUM_CORES * NUM_SUBCORES
SEQ_PER_WORKER = BATCH // NUM_WORKERS
HALF = SEQ_LENGTH // 2
LANES = 16
VPR = EMBED_DIM // LANES


def _body(idx_hbm, word_hbm, pos_hbm, out_hbm, idx_v, pos_v, rows_v, sem):
    c = lax.axis_index("c")
    s = lax.axis_index("s")
    wid = s * NUM_CORES + c

    pltpu.sync_copy(pos_hbm, pos_v)

    def seq_body(i, carry):
        seq = wid * SEQ_PER_WORKER + i
        pltpu.sync_copy(idx_hbm.at[seq], idx_v)
        cp0 = pltpu.async_copy(
            word_hbm.at[idx_v.at[0]], rows_v.at[pl.ds(0, HALF)], sem)
        cp1 = pltpu.async_copy(
            word_hbm.at[idx_v.at[1]], rows_v.at[pl.ds(HALF, HALF)], sem)
        cp0.wait()
        cp1.wait()

        def tok_body(t, carry2):
            for j in range(VPR):
                sl = pl.ds(j * LANES, LANES)
                plsc.addupdate(rows_v.at[t, sl], pos_v[t, sl])
            return carry2

        lax.fori_loop(0, SEQ_LENGTH, tok_body, 0, unroll=2)
        pltpu.sync_copy(rows_v, out_hbm.at[seq])
        return carry

    lax.fori_loop(0, SEQ_PER_WORKER, seq_body, 0)


@jax.jit
def kernel(inputs, word_table, pos_table):
    idx = inputs.reshape(BATCH, 2, HALF).astype(jnp.int32)
    mesh = plsc.VectorSubcoreMesh(
        core_axis_name="c", subcore_axis_name="s")
    run = pl.kernel(
        _body,
        out_type=jax.ShapeDtypeStruct((BATCH, SEQ_LENGTH, EMBED_DIM),
                                      jnp.float32),
        mesh=mesh,
        scratch_types=[
            pltpu.VMEM((2, HALF), jnp.int32),
            pltpu.VMEM((SEQ_LENGTH, EMBED_DIM), jnp.float32),
            pltpu.VMEM((SEQ_LENGTH, EMBED_DIM), jnp.float32),
            pltpu.SemaphoreType.DMA,
        ],
        compiler_params=pltpu.CompilerParams(use_tc_tiling_on_sc=False),
    )
    return run(idx, word_table, pos_table)


# speedup vs baseline: 3.1730x; 1.0352x over previous
import functools

import jax
import jax.numpy as jnp
from jax import lax
from jax.experimental import pallas as pl
from jax.experimental.pallas import tpu as pltpu
from jax.experimental.pallas import tpu_sc as plsc

MAX_WORD = 100000
EMBED_DIM = 64
SEQ_LENGTH = 200
BATCH = 4096

NUM_CORES = 2
NUM_SUBCORES = 16
NUM_WORKERS = NUM_CORES * NUM_SUBCORES
SEQ_PER_WORKER = BATCH // NUM_WORKERS
HALF = SEQ_LENGTH // 2
LANES = 16
VPR = EMBED_DIM // LANES


def _body(idx_hbm, word_hbm, pos_hbm, out_hbm, idx_v, pos_sh, rows_v, sem):
    c = lax.axis_index("c")
    s = lax.axis_index("s")
    wid = s * NUM_CORES + c

    @pl.when(s == 0)
    def _():
        pltpu.sync_copy(pos_hbm, pos_sh)

    plsc.subcore_barrier()

    def seq_body(i, carry):
        seq = wid * SEQ_PER_WORKER + i
        pltpu.sync_copy(idx_hbm.at[seq], idx_v)
        pltpu.sync_copy(pos_sh, rows_v)
        cp0 = pltpu.async_copy(
            word_hbm.at[idx_v.at[0]], rows_v.at[pl.ds(0, HALF)], sem,
            add=True)
        cp1 = pltpu.async_copy(
            word_hbm.at[idx_v.at[1]], rows_v.at[pl.ds(HALF, HALF)], sem,
            add=True)
        cp0.wait()
        cp1.wait()
        pltpu.sync_copy(rows_v, out_hbm.at[seq])
        return carry

    lax.fori_loop(0, SEQ_PER_WORKER, seq_body, 0)


@jax.jit
def kernel(inputs, word_table, pos_table):
    idx = inputs.reshape(BATCH, 2, HALF).astype(jnp.int32)
    mesh = plsc.VectorSubcoreMesh(
        core_axis_name="c", subcore_axis_name="s")
    run = pl.kernel(
        _body,
        out_type=jax.ShapeDtypeStruct((BATCH, SEQ_LENGTH, EMBED_DIM),
                                      jnp.float32),
        mesh=mesh,
        scratch_types=[
            pltpu.VMEM((2, HALF), jnp.int32),
            pltpu.VMEM_SHARED((SEQ_LENGTH, EMBED_DIM), jnp.float32),
            pltpu.VMEM((SEQ_LENGTH, EMBED_DIM), jnp.float32),
            pltpu.SemaphoreType.DMA,
        ],
        compiler_params=pltpu.CompilerParams(use_tc_tiling_on_sc=False),
    )
    return run(idx, word_table, pos_table)


# speedup vs baseline: 4.1431x; 1.3057x over previous
import jax
import jax.numpy as jnp
from jax import lax
from jax.experimental import pallas as pl
from jax.experimental.pallas import tpu as pltpu
from jax.experimental.pallas import tpu_sc as plsc

EMBED_DIM = 64
SEQ_LENGTH = 200
BATCH = 4096

NUM_CORES = 2
NUM_SUBCORES = 16
NUM_WORKERS = NUM_CORES * NUM_SUBCORES
SEQ_PER_WORKER = BATCH // NUM_WORKERS
HALF = SEQ_LENGTH // 2
NBUF = 4
GROUPS = SEQ_PER_WORKER // NBUF


def _body(idx_hbm, word_hbm, pos_hbm, out_hbm, idx_all, pos_sh, rows_v,
          *sems):
    sem_p = sems[0:NBUF]
    sem_g = sems[NBUF:2 * NBUF]
    sem_o = sems[2 * NBUF:3 * NBUF]
    c = lax.axis_index("c")
    s = lax.axis_index("s")
    wid = s * NUM_CORES + c
    base = wid * SEQ_PER_WORKER

    pltpu.sync_copy(idx_hbm.at[wid], idx_all)

    @pl.when(s == 0)
    def _():
        pltpu.sync_copy(pos_hbm, pos_sh)

    plsc.subcore_barrier()

    def group_body(g, carry):
        for b in range(NBUF):
            @pl.when(g > 0)
            def _(b=b):
                pltpu.make_async_copy(
                    rows_v.at[b], out_hbm.at[base], sem_o[b]).wait()
            pltpu.async_copy(pos_sh, rows_v.at[b], sem_p[b])
        for b in range(NBUF):
            i = g * NBUF + b
            pltpu.make_async_copy(pos_sh, rows_v.at[b], sem_p[b]).wait()
            pltpu.async_copy(
                word_hbm.at[idx_all.at[i, 0]],
                rows_v.at[b, pl.ds(0, HALF)], sem_g[b], add=True)
            pltpu.async_copy(
                word_hbm.at[idx_all.at[i, 1]],
                rows_v.at[b, pl.ds(HALF, HALF)], sem_g[b], add=True)
        for b in range(NBUF):
            i = g * NBUF + b
            pltpu.make_async_copy(
                word_hbm.at[idx_all.at[i, 0]],
                rows_v.at[b, pl.ds(0, HALF)], sem_g[b]).wait()
            pltpu.make_async_copy(
                word_hbm.at[idx_all.at[i, 1]],
                rows_v.at[b, pl.ds(HALF, HALF)], sem_g[b]).wait()
            pltpu.async_copy(rows_v.at[b], out_hbm.at[base + i], sem_o[b])
        return carry

    lax.fori_loop(0, GROUPS, group_body, 0)
    for b in range(NBUF):
        pltpu.make_async_copy(
            rows_v.at[b], out_hbm.at[base], sem_o[b]).wait()


@jax.jit
def kernel(inputs, word_table, pos_table):
    idx = inputs.reshape(NUM_WORKERS, SEQ_PER_WORKER, 2, HALF).astype(
        jnp.int32)
    mesh = plsc.VectorSubcoreMesh(
        core_axis_name="c", subcore_axis_name="s")
    run = pl.kernel(
        _body,
        out_type=jax.ShapeDtypeStruct((BATCH, SEQ_LENGTH, EMBED_DIM),
                                      jnp.float32),
        mesh=mesh,
        scratch_types=[
            pltpu.VMEM((SEQ_PER_WORKER, 2, HALF), jnp.int32),
            pltpu.VMEM_SHARED((SEQ_LENGTH, EMBED_DIM), jnp.float32),
            pltpu.VMEM((NBUF, SEQ_LENGTH, EMBED_DIM), jnp.float32),
        ] + [pltpu.SemaphoreType.DMA] * (3 * NBUF),
        compiler_params=pltpu.CompilerParams(use_tc_tiling_on_sc=False),
    )
    return run(idx, word_table, pos_table)
